# baseline (speedup 1.0000x reference)
import functools

import jax
import jax.numpy as jnp
from jax import lax
from jax.experimental import pallas as pl
from jax.experimental.pallas import tpu as pltpu
from jax.experimental.pallas import tpu_sc as plsc

B = 16384
L = 20
D = 16
LANES = 16


def _allsum(v):
    lane = lax.iota(jnp.int32, LANES)
    dn = lax.GatherDimensionNumbers(
        offset_dims=(), collapsed_slice_dims=(0,), start_index_map=(0,))
    for sh in (8, 4, 2, 1):
        perm = lax.gather(v, (lane ^ sh)[:, None], dn, slice_sizes=(1,),
                          mode=lax.GatherScatterMode.PROMISE_IN_BOUNDS)
        v = v + perm
    return v


def _rsqrt_vec(x):
    i = lax.bitcast_convert_type(x, jnp.int32)
    i = jnp.int32(0x5F3759DF) - lax.shift_right_logical(i, 1)
    y = lax.bitcast_convert_type(i, jnp.float32)
    y = y * (1.5 - 0.5 * x * y * y)
    y = y * (1.5 - 0.5 * x * y * y)
    return y


def _make_encoder(nc, ns):
    nw = nc * ns
    pw = B // nw
    ch = 128
    chunks = pw // ch
    rows_per_chunk = ch * L

    mesh = plsc.VectorSubcoreMesh(core_axis_name="c", subcore_axis_name="s")

    @functools.partial(
        pl.kernel,
        out_type=jax.ShapeDtypeStruct((B, D), jnp.float32),
        mesh=mesh,
        compiler_params=pltpu.CompilerParams(use_tc_tiling_on_sc=False),
        scratch_types=[
            pltpu.VMEM((pw, L), jnp.int32),
            pltpu.VMEM((2, rows_per_chunk, D), jnp.float32),
            pltpu.VMEM((ch, D), jnp.float32),
            pltpu.SemaphoreType.DMA((2,)),
        ],
    )
    def encode(idx_hbm, table_hbm, out_hbm, idx_v, rows_v, out_v, sem):
        wid = lax.axis_index("s") * nc + lax.axis_index("c")
        base_pt0 = pl.multiple_of(wid * pw, 8)
        pltpu.sync_copy(idx_hbm.at[pl.ds(base_pt0, pw)], idx_v)

        def issue(c):
            buf = c % 2

            def gather_one(p, carry):
                dst = rows_v.at[buf, pl.ds(pl.multiple_of(p * L, 4), L)]
                pltpu.async_copy(
                    table_hbm.at[idx_v.at[c * ch + p]], dst, sem.at[buf])
                return carry

            lax.fori_loop(0, ch, gather_one, 0)

        def drain(c):
            buf = c % 2
            pltpu.make_async_copy(
                table_hbm.at[pl.ds(0, rows_per_chunk)], rows_v.at[buf],
                sem.at[buf]).wait()

        issue(0)
        for c in range(chunks):
            if c + 1 < chunks:
                issue(c + 1)
            drain(c)
            buf = c % 2

            def point_body(p, carry, buf=buf):
                rbase = p * L
                acc = jnp.zeros((LANES,), jnp.float32)
                for j in range(L):
                    v = rows_v[buf, rbase + j]
                    acc = acc + v * _rsqrt_vec(_allsum(v * v))
                s2 = _allsum(acc * acc)
                out_v[p] = acc * _rsqrt_vec(s2)
                return carry

            lax.fori_loop(0, ch, point_body, 0)
            base_pt = pl.multiple_of(wid * pw + c * ch, 8)
            pltpu.sync_copy(out_v, out_hbm.at[pl.ds(base_pt, ch)])

    return encode


def kernel(indices, table):
    info = plsc.get_sparse_core_info()
    enc = _make_encoder(info.num_cores, info.num_subcores)
    return enc(indices.astype(jnp.int32), table)
